# Initial kernel scaffold; baseline (speedup 1.0000x reference)
import jax
import jax.numpy as jnp
from jax.experimental import pallas as pl


def kernel(xyz, points, W0, b0, g0, be0, W1, b1, g1, be1, W2, b2, g2, be2):
    raise NotImplementedError("write your pallas kernel here")


# speedup vs baseline: 15.3472x; 15.3472x over previous
import functools

import jax
import jax.numpy as jnp
from jax import lax
from jax.experimental import pallas as pl
from jax.experimental.pallas import tpu as pltpu
from jax.experimental.pallas import tpu_sc as plsc

_NPOINT = 512
_RADIUS = 0.2
_NSAMPLE = 32

_SC_NC = 2
_SC_NS = 16
_SC_NW = _SC_NC * _SC_NS
_GCHUNK = 128
_FDIM = 128


def _fps_body(xyz_ref, nx_ref):
    B, _, N = xyz_ref.shape
    iota = lax.broadcasted_iota(jnp.int32, (B, N), 1)
    io_s = lax.broadcasted_iota(jnp.int32, (B, _NPOINT), 1)
    x0 = xyz_ref[:, 0, :]
    x1 = xyz_ref[:, 1, :]
    x2 = xyz_ref[:, 2, :]

    def body(i, carry):
        dist_acc, far, nx0, nx1, nx2 = carry
        oh = (iota == far).astype(jnp.float32)
        c0 = jnp.sum(x0 * oh, axis=1, keepdims=True)
        c1 = jnp.sum(x1 * oh, axis=1, keepdims=True)
        c2 = jnp.sum(x2 * oh, axis=1, keepdims=True)
        hit = io_s == i
        nx0 = jnp.where(hit, c0, nx0)
        nx1 = jnp.where(hit, c1, nx1)
        nx2 = jnp.where(hit, c2, nx2)
        d0 = x0 - c0
        d1 = x1 - c1
        d2 = x2 - c2
        d = (d0 * d0 + d1 * d1) + d2 * d2
        dist_acc = jnp.minimum(dist_acc, d)
        m = jnp.max(dist_acc, axis=1, keepdims=True)
        sel = jnp.where(dist_acc == m, iota, N)
        far = jnp.min(sel, axis=1, keepdims=True).astype(jnp.int32)
        return dist_acc, far, nx0, nx1, nx2

    zs = jnp.zeros((B, _NPOINT), dtype=jnp.float32)
    init = (jnp.full((B, N), 1e10, dtype=jnp.float32),
            jnp.zeros((B, 1), dtype=jnp.int32), zs, zs, zs)
    _, _, nx0, nx1, nx2 = lax.fori_loop(0, _NPOINT, body, init)
    nx_ref[:, 0, :] = nx0
    nx_ref[:, 1, :] = nx1
    nx_ref[:, 2, :] = nx2


def _fps(xyz):
    B, _, N = xyz.shape
    return pl.pallas_call(
        _fps_body,
        out_shape=jax.ShapeDtypeStruct((B, 3, _NPOINT), jnp.float32),
    )(xyz)


def _bq_body(xyz_ref, nxp_ref, gid_ref):
    _, _, N = xyz_ref.shape
    S = nxp_ref.shape[1]
    b = pl.program_id(0)
    x0 = xyz_ref[:, 0, :]
    x1 = xyz_ref[:, 1, :]
    x2 = xyz_ref[:, 2, :]
    cb = nxp_ref[0]
    c0 = cb[:, 0:1]
    c1 = cb[:, 1:2]
    c2 = cb[:, 2:3]
    sqx = (x0 * x0 + x1 * x1) + x2 * x2
    sqc = (c0 * c0 + c1 * c1) + c2 * c2
    bfr = lambda v: v.astype(jnp.bfloat16).astype(jnp.float32)
    dot = (bfr(c0) * bfr(x0) + bfr(c1) * bfr(x1)) + bfr(c2) * bfr(x2)
    sqd = sqc + sqx - 2.0 * dot
    mask = sqd <= jnp.float32(_RADIUS * _RADIUS)
    iota = lax.broadcasted_iota(jnp.int32, (1, N), 1)
    iota = jnp.broadcast_to(iota, mask.shape)
    big = jnp.int32(N)
    base = b * jnp.int32(N)

    prev = jnp.full((S, 1), -1, dtype=jnp.int32)
    first = None
    cols = []
    for k in range(_NSAMPLE):
        cand = jnp.where(mask & (iota > prev), iota, big)
        cur = jnp.min(cand, axis=1, keepdims=True)
        if first is None:
            first = cur
        cols.append(jnp.where(cur >= big, first, cur) + base)
        prev = cur
    gid_ref[0] = jnp.concatenate(cols, axis=1)


def _ballquery(xyz, nxp):
    B, _, N = xyz.shape
    S = nxp.shape[1]
    return pl.pallas_call(
        _bq_body,
        grid=(B,),
        in_specs=[
            pl.BlockSpec((1, 3, N), lambda b: (b, 0, 0)),
            pl.BlockSpec((1, S, 16), lambda b: (b, 0, 0)),
        ],
        out_specs=pl.BlockSpec((1, S, _NSAMPLE), lambda b: (b, 0, 0)),
        out_shape=jax.ShapeDtypeStruct((B, S, _NSAMPLE), jnp.int32),
    )(xyz, nxp)


def _sc_gather(table, ids3):
    R = ids3.shape[0] * ids3.shape[1] * ids3.shape[2]
    rows_per_w = R // _SC_NW
    nchunk = rows_per_w // _GCHUNK
    mesh = plsc.VectorSubcoreMesh(core_axis_name="c", subcore_axis_name="s")

    @functools.partial(
        pl.kernel,
        mesh=mesh,
        out_type=jax.ShapeDtypeStruct((R, _FDIM), jnp.float32),
        scratch_types=[
            pltpu.VMEM((nchunk, _GCHUNK), jnp.int32),
            pltpu.VMEM((_GCHUNK, _FDIM), jnp.float32),
            pltpu.SemaphoreType.DMA,
        ],
    )
    def k(table_hbm, ids_hbm, out_hbm, idx_v, rows_v, sem):
        wid = lax.axis_index("s") * _SC_NC + lax.axis_index("c")
        pltpu.sync_copy(ids_hbm.at[wid], idx_v)
        base = wid * rows_per_w

        def chunk(j, carry):
            pltpu.async_copy(table_hbm.at[idx_v.at[j]], rows_v, sem).wait()
            pltpu.sync_copy(rows_v,
                            out_hbm.at[pl.ds(base + j * _GCHUNK, _GCHUNK)])
            return carry

        lax.fori_loop(0, nchunk, chunk, 0)

    return k(table, ids3)


def _mlp_body(g_ref, cb_ref, w1_ref, w1x_ref, b1_ref, w2_ref, b2_ref,
              w3_ref, b3_ref, out_ref):
    SB = out_ref.shape[0]
    K = _NSAMPLE
    g = g_ref[...]
    off = jnp.dot(cb_ref[...], w1x_ref[...],
                  preferred_element_type=jnp.float32)
    h = jnp.dot(g, w1_ref[...], preferred_element_type=jnp.float32)
    h = h + b1_ref[...]
    h = h.reshape(SB, K, h.shape[-1]) - off[:, None, :]
    h = jnp.maximum(h, 0.0).reshape(SB * K, -1)
    h = jnp.dot(h, w2_ref[...], preferred_element_type=jnp.float32)
    h = jnp.maximum(h + b2_ref[...], 0.0)
    h = jnp.dot(h, w3_ref[...], preferred_element_type=jnp.float32)
    h = jnp.maximum(h + b3_ref[...], 0.0)
    out_ref[...] = jnp.max(h.reshape(SB, K, h.shape[-1]), axis=1)


def _mlp(grows, cbflat, w1t, w1xt, b1, w2t, b2, w3t, b3, sblk=64):
    R = grows.shape[0]
    BS = cbflat.shape[0]
    nblk = BS // sblk
    co = w3t.shape[1]
    full = lambda shape: pl.BlockSpec(shape, lambda r: tuple(0 for _ in shape))
    return pl.pallas_call(
        _mlp_body,
        grid=(nblk,),
        in_specs=[
            pl.BlockSpec((sblk * _NSAMPLE, _FDIM), lambda r: (r, 0)),
            pl.BlockSpec((sblk, 16), lambda r: (r, 0)),
            full(w1t.shape),
            full(w1xt.shape),
            full(b1.shape),
            full(w2t.shape),
            full(b2.shape),
            full(w3t.shape),
            full(b3.shape),
        ],
        out_specs=pl.BlockSpec((sblk, co), lambda r: (r, 0)),
        out_shape=jax.ShapeDtypeStruct((BS, co), jnp.float32),
    )(grows, cbflat, w1t, w1xt, b1, w2t, b2, w3t, b3)


def kernel(xyz, points, W0, b0, g0, be0, W1, b1, g1, be1, W2, b2, g2, be2):
    B, _, N = xyz.shape
    D = points.shape[1]
    S = _NPOINT

    inv = 1.0 / jnp.sqrt(jnp.float32(1.0 + 1e-5))
    def fold(W, b, g, be):
        sc = inv * g
        return W * sc[:, None], b * sc + be
    Wf0, bf0 = fold(W0, b0, g0, be0)
    Wf1, bf1 = fold(W1, b1, g1, be1)
    Wf2, bf2 = fold(W2, b2, g2, be2)

    nx = _fps(xyz)

    nxt = jnp.transpose(nx, (0, 2, 1))
    nxp = jnp.concatenate(
        [nxt, jnp.zeros((B, S, 13), jnp.float32)], axis=-1)

    gids = _ballquery(xyz, nxp)
    ids3 = gids.reshape(_SC_NW, -1, _GCHUNK)

    xyz_t = jnp.transpose(xyz, (0, 2, 1))
    pts_t = jnp.transpose(points, (0, 2, 1))
    table = jnp.concatenate(
        [xyz_t, pts_t, jnp.zeros((B, N, _FDIM - 3 - D), jnp.float32)],
        axis=-1).reshape(B * N, _FDIM)

    grows = _sc_gather(table, ids3)

    w1t = jnp.zeros((_FDIM, Wf0.shape[0]), jnp.float32)
    w1t = w1t.at[:3 + D, :].set(Wf0.T)
    w1xt = jnp.zeros((16, Wf0.shape[0]), jnp.float32)
    w1xt = w1xt.at[:3, :].set(Wf0[:, :3].T)

    out = _mlp(grows, nxp.reshape(B * S, 16),
               w1t, w1xt, bf0.reshape(1, -1),
               Wf1.T, bf1.reshape(1, -1),
               Wf2.T, bf2.reshape(1, -1))

    new_points = jnp.transpose(out.reshape(B, S, -1), (0, 2, 1))
    return nx, new_points


# speedup vs baseline: 17.3208x; 1.1286x over previous
import functools

import jax
import jax.numpy as jnp
from jax import lax
from jax.experimental import pallas as pl
from jax.experimental.pallas import tpu as pltpu
from jax.experimental.pallas import tpu_sc as plsc

_NPOINT = 512
_RADIUS = 0.2
_NSAMPLE = 32

_SC_NC = 2
_SC_NS = 16
_SC_NW = _SC_NC * _SC_NS
_GCHUNK = 128
_TDIM = 128
_FDIM = 128
_NBUF = 4


def _fps_body(xyz_ref, nx_ref):
    B, _, N = xyz_ref.shape
    iota = lax.broadcasted_iota(jnp.int32, (B, N), 1)
    io_s = lax.broadcasted_iota(jnp.int32, (B, _NPOINT), 1)
    x0 = xyz_ref[:, 0, :]
    x1 = xyz_ref[:, 1, :]
    x2 = xyz_ref[:, 2, :]

    def body(i, carry):
        dist_acc, far, nx0, nx1, nx2 = carry
        oh = (iota == far).astype(jnp.float32)
        c0 = jnp.sum(x0 * oh, axis=1, keepdims=True)
        c1 = jnp.sum(x1 * oh, axis=1, keepdims=True)
        c2 = jnp.sum(x2 * oh, axis=1, keepdims=True)
        hit = io_s == i
        nx0 = jnp.where(hit, c0, nx0)
        nx1 = jnp.where(hit, c1, nx1)
        nx2 = jnp.where(hit, c2, nx2)
        d0 = x0 - c0
        d1 = x1 - c1
        d2 = x2 - c2
        d = (d0 * d0 + d1 * d1) + d2 * d2
        dist_acc = jnp.minimum(dist_acc, d)
        m = jnp.max(dist_acc, axis=1, keepdims=True)
        sel = jnp.where(dist_acc == m, iota, N)
        far = jnp.min(sel, axis=1, keepdims=True).astype(jnp.int32)
        return dist_acc, far, nx0, nx1, nx2

    zs = jnp.zeros((B, _NPOINT), dtype=jnp.float32)
    init = (jnp.full((B, N), 1e10, dtype=jnp.float32),
            jnp.zeros((B, 1), dtype=jnp.int32), zs, zs, zs)
    _, _, nx0, nx1, nx2 = lax.fori_loop(0, _NPOINT, body, init)
    nx_ref[:, 0, :] = nx0
    nx_ref[:, 1, :] = nx1
    nx_ref[:, 2, :] = nx2


def _fps(xyz):
    B, _, N = xyz.shape
    return pl.pallas_call(
        _fps_body,
        out_shape=jax.ShapeDtypeStruct((B, 3, _NPOINT), jnp.float32),
    )(xyz)


def _bq_body(xyz_ref, nxp_ref, gid_ref):
    _, _, N = xyz_ref.shape
    S = nxp_ref.shape[1]
    b = pl.program_id(0)
    x0 = xyz_ref[:, 0, :]
    x1 = xyz_ref[:, 1, :]
    x2 = xyz_ref[:, 2, :]
    cb = nxp_ref[0]
    c0 = cb[:, 0:1]
    c1 = cb[:, 1:2]
    c2 = cb[:, 2:3]
    sqx = (x0 * x0 + x1 * x1) + x2 * x2
    sqc = (c0 * c0 + c1 * c1) + c2 * c2
    bfr = lambda v: v.astype(jnp.bfloat16).astype(jnp.float32)
    dot = (bfr(c0) * bfr(x0) + bfr(c1) * bfr(x1)) + bfr(c2) * bfr(x2)
    sqd = sqc + sqx - 2.0 * dot
    mask = sqd <= jnp.float32(_RADIUS * _RADIUS)
    iota = lax.broadcasted_iota(jnp.int32, (1, N), 1)
    iota = jnp.broadcast_to(iota, mask.shape)
    big = jnp.int32(N)
    base = b * jnp.int32(N)

    mio = jnp.where(mask, iota, big)
    first = jnp.min(mio, axis=1, keepdims=True)
    prev = first
    cols = [first + base]
    for k in range(1, _NSAMPLE):
        cand = jnp.where(mio > prev, mio, big)
        cur = jnp.min(cand, axis=1, keepdims=True)
        cols.append(jnp.where(cur >= big, first, cur) + base)
        prev = cur
    gid_ref[0] = jnp.concatenate(cols, axis=1)


def _ballquery(xyz, nxp):
    B, _, N = xyz.shape
    S = nxp.shape[1]
    return pl.pallas_call(
        _bq_body,
        grid=(B,),
        in_specs=[
            pl.BlockSpec((1, 3, N), lambda b: (b, 0, 0)),
            pl.BlockSpec((1, S, 16), lambda b: (b, 0, 0)),
        ],
        out_specs=pl.BlockSpec((1, S, _NSAMPLE), lambda b: (b, 0, 0)),
        out_shape=jax.ShapeDtypeStruct((B, S, _NSAMPLE), jnp.int32),
    )(xyz, nxp)


def _sc_gather(table, ids3):
    R = ids3.shape[0] * ids3.shape[1] * ids3.shape[2]
    rows_per_w = R // _SC_NW
    nchunk = rows_per_w // _GCHUNK
    nrounds = nchunk // _NBUF
    mesh = plsc.VectorSubcoreMesh(core_axis_name="c", subcore_axis_name="s")

    @functools.partial(
        pl.kernel,
        mesh=mesh,
        out_type=jax.ShapeDtypeStruct((R, _FDIM), jnp.float32),
        scratch_types=[
            pltpu.VMEM((nchunk, _GCHUNK), jnp.int32),
            [pltpu.VMEM((_GCHUNK, _TDIM), jnp.float32)] * _NBUF,
            pltpu.SemaphoreType.DMA((_NBUF,)),
            pltpu.SemaphoreType.DMA((_NBUF,)),
        ],
    )
    def k(table_hbm, ids_hbm, out_hbm, idx_v, bufs, gsem, osem):
        wid = lax.axis_index("s") * _SC_NC + lax.axis_index("c")
        pltpu.sync_copy(ids_hbm.at[wid], idx_v)
        base = wid * rows_per_w

        def g_copy(j, b):
            return pltpu.make_async_copy(
                table_hbm.at[idx_v.at[j]], bufs[b], gsem.at[b])

        def o_copy(j, b):
            return pltpu.make_async_copy(
                bufs[b],
                out_hbm.at[pl.ds(base + j * _GCHUNK, _GCHUNK)], osem.at[b])

        for b in range(_NBUF):
            g_copy(b, b).start()

        def round_(t, carry):
            for b in range(_NBUF):
                j = t * _NBUF + b
                g_copy(j, b).wait()
                o_copy(j, b).start()
            for b in range(_NBUF):
                j = t * _NBUF + b
                o_copy(j, b).wait()

                @pl.when(t < nrounds - 1)
                def _(b=b, j=j):
                    g_copy(j + _NBUF, b).start()
            return carry

        lax.fori_loop(0, nrounds, round_, 0)

    return k(table, ids3)


def _mlp_body(g_ref, cb_ref, w1_ref, w1x_ref, b1_ref, w2_ref, b2_ref,
              w3_ref, b3_ref, out_ref):
    SB = out_ref.shape[0]
    K = _NSAMPLE
    g = g_ref[...]
    off = jnp.dot(cb_ref[...], w1x_ref[...],
                  preferred_element_type=jnp.float32)
    h = jnp.dot(g, w1_ref[...], preferred_element_type=jnp.float32)
    h = h + b1_ref[...]
    h = h.reshape(SB, K, h.shape[-1]) - off[:, None, :]
    h = jnp.maximum(h, 0.0).reshape(SB * K, -1)
    h = jnp.dot(h, w2_ref[...], preferred_element_type=jnp.float32)
    h = jnp.maximum(h + b2_ref[...], 0.0)
    h = jnp.dot(h, w3_ref[...], preferred_element_type=jnp.float32)
    h = jnp.maximum(h + b3_ref[...], 0.0)
    out_ref[...] = jnp.max(h.reshape(SB, K, h.shape[-1]), axis=1)


def _mlp(grows, cbflat, w1t, w1xt, b1, w2t, b2, w3t, b3, sblk=64):
    R = grows.shape[0]
    BS = cbflat.shape[0]
    nblk = BS // sblk
    co = w3t.shape[1]
    full = lambda shape: pl.BlockSpec(shape, lambda r: tuple(0 for _ in shape))
    return pl.pallas_call(
        _mlp_body,
        grid=(nblk,),
        in_specs=[
            pl.BlockSpec((sblk * _NSAMPLE, _FDIM), lambda r: (r, 0)),
            pl.BlockSpec((sblk, 16), lambda r: (r, 0)),
            full(w1t.shape),
            full(w1xt.shape),
            full(b1.shape),
            full(w2t.shape),
            full(b2.shape),
            full(w3t.shape),
            full(b3.shape),
        ],
        out_specs=pl.BlockSpec((sblk, co), lambda r: (r, 0)),
        out_shape=jax.ShapeDtypeStruct((BS, co), jnp.float32),
    )(grows, cbflat, w1t, w1xt, b1, w2t, b2, w3t, b3)


def kernel(xyz, points, W0, b0, g0, be0, W1, b1, g1, be1, W2, b2, g2, be2):
    B, _, N = xyz.shape
    D = points.shape[1]
    S = _NPOINT

    inv = 1.0 / jnp.sqrt(jnp.float32(1.0 + 1e-5))
    def fold(W, b, g, be):
        sc = inv * g
        return W * sc[:, None], b * sc + be
    Wf0, bf0 = fold(W0, b0, g0, be0)
    Wf1, bf1 = fold(W1, b1, g1, be1)
    Wf2, bf2 = fold(W2, b2, g2, be2)

    nx = _fps(xyz)

    nxt = jnp.transpose(nx, (0, 2, 1))
    nxp = jnp.concatenate(
        [nxt, jnp.zeros((B, S, 13), jnp.float32)], axis=-1)

    gids = _ballquery(xyz, nxp)
    ids3 = gids.reshape(_SC_NW, -1, _GCHUNK)

    xyz_t = jnp.transpose(xyz, (0, 2, 1))
    pts_t = jnp.transpose(points, (0, 2, 1))
    table = jnp.concatenate(
        [xyz_t, pts_t, jnp.zeros((B, N, _TDIM - 3 - D), jnp.float32)],
        axis=-1).reshape(B * N, _TDIM)

    grows = _sc_gather(table, ids3)

    w1t = jnp.zeros((_FDIM, Wf0.shape[0]), jnp.float32)
    w1t = w1t.at[:3 + D, :].set(Wf0.T)
    w1xt = jnp.zeros((16, Wf0.shape[0]), jnp.float32)
    w1xt = w1xt.at[:3, :].set(Wf0[:, :3].T)

    out = _mlp(grows, nxp.reshape(B * S, 16),
               w1t, w1xt, bf0.reshape(1, -1),
               Wf1.T, bf1.reshape(1, -1),
               Wf2.T, bf2.reshape(1, -1))

    new_points = jnp.transpose(out.reshape(B, S, -1), (0, 2, 1))
    return nx, new_points
